# baseline (speedup 1.0000x reference)
import functools

import jax
import jax.numpy as jnp
from jax import lax
from jax.experimental import pallas as pl
from jax.experimental.pallas import tpu as pltpu
from jax.experimental.pallas import tpu_sc as plsc

B = 64
N = 20000
ROWS = B * N
NUM_WORKERS = 32
ROWS_PER_WORKER = ROWS // NUM_WORKERS
CHUNK_ROWS = 2000
NUM_CHUNKS = ROWS_PER_WORKER // CHUNK_ROWS
GROUPS = CHUNK_ROWS // 16
CHUNK_ELEMS = CHUNK_ROWS * 4


def _sc_body(c_hbm, t_hbm, p_hbm, out_hbm, c_v, t_v, p_v, acc_v):
    nc = 2
    wid = lax.axis_index("s") * nc + lax.axis_index("c")
    row_base = wid * ROWS_PER_WORKER
    elem_base = row_base * 4

    pat = lax.shift_right_logical(lax.iota(jnp.int32, 16), 2)
    zero = jnp.zeros((16,), jnp.float32)

    def chunk_body(k, carry):
        sq, wa = carry
        c_off = pl.multiple_of(row_base + k * CHUNK_ROWS, 8)
        e_off = pl.multiple_of(elem_base + k * CHUNK_ELEMS, 8)
        pltpu.sync_copy(c_hbm.at[pl.ds(c_off, CHUNK_ROWS)], c_v)
        pltpu.sync_copy(t_hbm.at[pl.ds(e_off, CHUNK_ELEMS)], t_v)
        pltpu.sync_copy(p_hbm.at[pl.ds(e_off, CHUNK_ELEMS)], p_v)

        def group_body(g, gcarry):
            gsq, gwa = gcarry
            c16 = c_v[pl.ds(g * 16, 16)]
            w = jnp.where(c16 != 0, jnp.float32(1), jnp.float32(0))
            gwa = gwa + w
            for j in range(4):
                w16 = w.at[(j * 4) + pat].get(mode="promise_in_bounds")
                t16 = t_v[pl.ds(g * 64 + j * 16, 16)]
                p16 = p_v[pl.ds(g * 64 + j * 16, 16)]
                d = p16 - t16
                gsq = gsq + d * d * w16
            return gsq, gwa

        return lax.fori_loop(0, GROUPS, group_body, (sq, wa))

    sq, wa = lax.fori_loop(0, NUM_CHUNKS, chunk_body, (zero, zero))

    acc_v[pl.ds(0, 16)] = sq
    acc_v[pl.ds(16, 16)] = wa
    pltpu.sync_copy(acc_v, out_hbm.at[wid])


@jax.jit
def _bbox_mse_sc(c_flat, t_flat, p_flat):
    mesh = plsc.VectorSubcoreMesh(core_axis_name="c", subcore_axis_name="s")
    partials = pl.kernel(
        _sc_body,
        mesh=mesh,
        out_type=jax.ShapeDtypeStruct((NUM_WORKERS, 32), jnp.float32),
        scratch_types=[
            pltpu.VMEM((CHUNK_ROWS,), jnp.int32),
            pltpu.VMEM((CHUNK_ELEMS,), jnp.float32),
            pltpu.VMEM((CHUNK_ELEMS,), jnp.float32),
            pltpu.VMEM((32,), jnp.float32),
        ],
    )(c_flat, t_flat, p_flat)
    num = jnp.sum(partials[:, :16])
    den = jnp.sum(partials[:, 16:]) * 4.0
    return num / den


def kernel(target_bbox, target_class, predicted_bbox):
    t_flat = target_bbox.reshape(-1)
    p_flat = predicted_bbox.reshape(-1)
    c_flat = target_class.reshape(-1).astype(jnp.int32)
    return _bbox_mse_sc(c_flat, t_flat, p_flat)


# speedup vs baseline: 52.0171x; 52.0171x over previous
import jax
import jax.numpy as jnp
from jax import lax
from jax.experimental import pallas as pl
from jax.experimental.pallas import tpu as pltpu
from jax.experimental.pallas import tpu_sc as plsc

B = 64
N = 20000
N_MAIN = 19968
N_TAIL = N - N_MAIN
NUM_WORKERS = 32
OCTET = 8
QUARTER = N_MAIN // 4
CHUNK_N = 1664
NUM_CHUNKS = QUARTER // CHUNK_N


def _sc_body(c_hbm, t_hbm, p_hbm, ct_hbm, tt_hbm, pt_hbm, out_hbm,
             c_v, t_v, p_v, ct_v, tt_v, pt_v, acc_v):
    nc = 2
    wid = lax.axis_index("s") * nc + lax.axis_index("c")
    o = wid // 4
    q = wid % 4
    b0 = pl.multiple_of(o * OCTET, 8)
    nbase = q * QUARTER
    zero = jnp.zeros((16,), jnp.float32)

    def chunk_body(k, carry):
        n0 = pl.multiple_of(nbase + k * CHUNK_N, 128)
        pltpu.sync_copy(c_hbm.at[pl.ds(b0, OCTET), pl.ds(n0, CHUNK_N)], c_v)
        pltpu.sync_copy(t_hbm.at[pl.ds(b0, OCTET), :, pl.ds(n0, CHUNK_N)], t_v)
        pltpu.sync_copy(p_hbm.at[pl.ds(b0, OCTET), :, pl.ds(n0, CHUNK_N)], p_v)

        def batch_body(bi, bcarry):
            def group_body(g, gcarry):
                gsq, gwa = gcarry
                c16 = c_v[bi, pl.ds(g * 16, 16)]
                w = jnp.where(c16 != 0, jnp.float32(1), jnp.float32(0))
                gwa = gwa + w
                for comp in range(4):
                    t16 = t_v[bi, comp, pl.ds(g * 16, 16)]
                    p16 = p_v[bi, comp, pl.ds(g * 16, 16)]
                    d = p16 - t16
                    gsq = gsq + d * d * w
                return gsq, gwa

            return lax.fori_loop(0, CHUNK_N // 16, group_body, bcarry)

        return lax.fori_loop(0, OCTET, batch_body, carry)

    carry = lax.fori_loop(0, NUM_CHUNKS, chunk_body, (zero, zero))

    tb = wid * 2 * 4 * N_TAIL
    cb = wid * 2 * N_TAIL
    pltpu.sync_copy(ct_hbm.at[pl.ds(cb, 2 * N_TAIL)], ct_v)
    pltpu.sync_copy(tt_hbm.at[pl.ds(tb, 8 * N_TAIL)], tt_v)
    pltpu.sync_copy(pt_hbm.at[pl.ds(tb, 8 * N_TAIL)], pt_v)

    def tail_body(i, carry):
        sq, wa = carry
        bi = i // 2
        h = i % 2
        c16 = ct_v[pl.ds(bi * N_TAIL + h * 16, 16)]
        w = jnp.where(c16 != 0, jnp.float32(1), jnp.float32(0))
        wa = wa + w
        for comp in range(4):
            off = bi * 4 * N_TAIL + comp * N_TAIL + h * 16
            d = pt_v[pl.ds(off, 16)] - tt_v[pl.ds(off, 16)]
            sq = sq + d * d * w
        return sq, wa

    sq, wa = lax.fori_loop(0, 4, tail_body, carry)

    acc_v[pl.ds(0, 16)] = sq
    acc_v[pl.ds(16, 16)] = wa
    pltpu.sync_copy(acc_v, out_hbm.at[wid])


@jax.jit
def _bbox_mse_sc(c2d, t3d, p3d, c_tail, t_tail, p_tail):
    mesh = plsc.VectorSubcoreMesh(core_axis_name="c", subcore_axis_name="s")
    partials = pl.kernel(
        _sc_body,
        mesh=mesh,
        out_type=jax.ShapeDtypeStruct((NUM_WORKERS, 32), jnp.float32),
        scratch_types=[
            pltpu.VMEM((OCTET, CHUNK_N), jnp.int32),
            pltpu.VMEM((OCTET, 4, CHUNK_N), jnp.float32),
            pltpu.VMEM((OCTET, 4, CHUNK_N), jnp.float32),
            pltpu.VMEM((2 * N_TAIL,), jnp.int32),
            pltpu.VMEM((8 * N_TAIL,), jnp.float32),
            pltpu.VMEM((8 * N_TAIL,), jnp.float32),
            pltpu.VMEM((32,), jnp.float32),
        ],
    )(c2d, t3d, p3d, c_tail, t_tail, p_tail)
    num = jnp.sum(partials[:, :16])
    den = jnp.sum(partials[:, 16:]) * 4.0
    return num / den


def kernel(target_bbox, target_class, predicted_bbox):
    t3d = jnp.transpose(target_bbox, (0, 2, 1))
    p3d = jnp.transpose(predicted_bbox, (0, 2, 1))
    c2d = target_class.astype(jnp.int32)
    t_tail = t3d[:, :, N_MAIN:].reshape(-1)
    p_tail = p3d[:, :, N_MAIN:].reshape(-1)
    c_tail = c2d[:, N_MAIN:].reshape(-1)
    return _bbox_mse_sc(c2d, t3d, p3d, c_tail, t_tail, p_tail)


# speedup vs baseline: 61.0045x; 1.1728x over previous
import jax
import jax.numpy as jnp
from jax import lax
from jax.experimental import pallas as pl
from jax.experimental.pallas import tpu as pltpu
from jax.experimental.pallas import tpu_sc as plsc

B = 64
N = 20000
N_MAIN = 19968
N_TAIL = N - N_MAIN
NUM_WORKERS = 32
OCTET = 8
QUARTER = N_MAIN // 4
CHUNK_N = 384
NUM_CHUNKS = QUARTER // CHUNK_N


def _sc_body(c_hbm, t_hbm, p_hbm, ct_hbm, tt_hbm, pt_hbm, out_hbm,
             c0, c1, t0, t1, p0, p1, ct_v, tt_v, pt_v, acc_v,
             sc0, sc1, st0, st1, sp0, sp1):
    nc = 2
    wid = lax.axis_index("s") * nc + lax.axis_index("c")
    o = wid // 4
    q = wid % 4
    b0 = pl.multiple_of(o * OCTET, 8)
    nbase = q * QUARTER
    zero = jnp.zeros((16,), jnp.float32)

    def start(k, cv, tv, pv, cs, ts, ps):
        n0 = pl.multiple_of(nbase + k * CHUNK_N, 128)
        pltpu.make_async_copy(
            c_hbm.at[pl.ds(b0, OCTET), pl.ds(n0, CHUNK_N)], cv, cs).start()
        pltpu.make_async_copy(
            t_hbm.at[pl.ds(b0, OCTET), :, pl.ds(n0, CHUNK_N)], tv, ts).start()
        pltpu.make_async_copy(
            p_hbm.at[pl.ds(b0, OCTET), :, pl.ds(n0, CHUNK_N)], pv, ps).start()

    def wait(k, cv, tv, pv, cs, ts, ps):
        n0 = pl.multiple_of(nbase + k * CHUNK_N, 128)
        pltpu.make_async_copy(
            c_hbm.at[pl.ds(b0, OCTET), pl.ds(n0, CHUNK_N)], cv, cs).wait()
        pltpu.make_async_copy(
            t_hbm.at[pl.ds(b0, OCTET), :, pl.ds(n0, CHUNK_N)], tv, ts).wait()
        pltpu.make_async_copy(
            p_hbm.at[pl.ds(b0, OCTET), :, pl.ds(n0, CHUNK_N)], pv, ps).wait()

    def compute(cv, tv, pv, carry):
        def batch_body(bi, bcarry):
            def group_body(g, gcarry):
                gsq, gwa = gcarry
                c16 = cv[bi, pl.ds(g * 16, 16)]
                w = jnp.where(c16 != 0, jnp.float32(1), jnp.float32(0))
                gwa = gwa + w
                d = pv[bi, 0, pl.ds(g * 16, 16)] - tv[bi, 0, pl.ds(g * 16, 16)]
                s = d * d
                for comp in range(1, 4):
                    d = (pv[bi, comp, pl.ds(g * 16, 16)]
                         - tv[bi, comp, pl.ds(g * 16, 16)])
                    s = s + d * d
                return gsq + s * w, gwa

            return lax.fori_loop(0, CHUNK_N // 16, group_body, bcarry)

        return lax.fori_loop(0, OCTET, batch_body, carry)

    start(0, c0, t0, p0, sc0, st0, sp0)

    def pair_body(i, carry):
        ka = 2 * i
        wait(ka, c0, t0, p0, sc0, st0, sp0)
        start(ka + 1, c1, t1, p1, sc1, st1, sp1)
        carry = compute(c0, t0, p0, carry)
        wait(ka + 1, c1, t1, p1, sc1, st1, sp1)
        start(ka + 2, c0, t0, p0, sc0, st0, sp0)
        return compute(c1, t1, p1, carry)

    carry = lax.fori_loop(0, NUM_CHUNKS // 2, pair_body, (zero, zero))
    wait(NUM_CHUNKS - 1, c0, t0, p0, sc0, st0, sp0)
    carry = compute(c0, t0, p0, carry)

    tb = wid * 2 * 4 * N_TAIL
    cb = wid * 2 * N_TAIL
    pltpu.sync_copy(ct_hbm.at[pl.ds(cb, 2 * N_TAIL)], ct_v)
    pltpu.sync_copy(tt_hbm.at[pl.ds(tb, 8 * N_TAIL)], tt_v)
    pltpu.sync_copy(pt_hbm.at[pl.ds(tb, 8 * N_TAIL)], pt_v)

    def tail_body(i, tcarry):
        sq, wa = tcarry
        bi = i // 2
        h = i % 2
        c16 = ct_v[pl.ds(bi * N_TAIL + h * 16, 16)]
        w = jnp.where(c16 != 0, jnp.float32(1), jnp.float32(0))
        wa = wa + w
        off = bi * 4 * N_TAIL + h * 16
        d = pt_v[pl.ds(off, 16)] - tt_v[pl.ds(off, 16)]
        s = d * d
        for comp in range(1, 4):
            off = bi * 4 * N_TAIL + comp * N_TAIL + h * 16
            d = pt_v[pl.ds(off, 16)] - tt_v[pl.ds(off, 16)]
            s = s + d * d
        return sq + s * w, wa

    sq, wa = lax.fori_loop(0, 4, tail_body, carry)

    acc_v[pl.ds(0, 16)] = sq
    acc_v[pl.ds(16, 16)] = wa
    pltpu.sync_copy(acc_v, out_hbm.at[wid])


@jax.jit
def _bbox_mse_sc(c2d, t3d, p3d, c_tail, t_tail, p_tail):
    mesh = plsc.VectorSubcoreMesh(core_axis_name="c", subcore_axis_name="s")
    partials = pl.kernel(
        _sc_body,
        mesh=mesh,
        out_type=jax.ShapeDtypeStruct((NUM_WORKERS, 32), jnp.float32),
        scratch_types=[
            pltpu.VMEM((OCTET, CHUNK_N), jnp.int32),
            pltpu.VMEM((OCTET, CHUNK_N), jnp.int32),
            pltpu.VMEM((OCTET, 4, CHUNK_N), jnp.float32),
            pltpu.VMEM((OCTET, 4, CHUNK_N), jnp.float32),
            pltpu.VMEM((OCTET, 4, CHUNK_N), jnp.float32),
            pltpu.VMEM((OCTET, 4, CHUNK_N), jnp.float32),
            pltpu.VMEM((2 * N_TAIL,), jnp.int32),
            pltpu.VMEM((8 * N_TAIL,), jnp.float32),
            pltpu.VMEM((8 * N_TAIL,), jnp.float32),
            pltpu.VMEM((32,), jnp.float32),
            pltpu.SemaphoreType.DMA,
            pltpu.SemaphoreType.DMA,
            pltpu.SemaphoreType.DMA,
            pltpu.SemaphoreType.DMA,
            pltpu.SemaphoreType.DMA,
            pltpu.SemaphoreType.DMA,
        ],
    )(c2d, t3d, p3d, c_tail, t_tail, p_tail)
    num = jnp.sum(partials[:, :16])
    den = jnp.sum(partials[:, 16:]) * 4.0
    return num / den


def kernel(target_bbox, target_class, predicted_bbox):
    t3d = jnp.transpose(target_bbox, (0, 2, 1))
    p3d = jnp.transpose(predicted_bbox, (0, 2, 1))
    c2d = target_class.astype(jnp.int32)
    t_tail = t3d[:, :, N_MAIN:].reshape(-1)
    p_tail = p3d[:, :, N_MAIN:].reshape(-1)
    c_tail = c2d[:, N_MAIN:].reshape(-1)
    return _bbox_mse_sc(c2d, t3d, p3d, c_tail, t_tail, p_tail)
